# baseline (speedup 1.0000x reference)
import functools

import jax
import jax.numpy as jnp
from jax import lax
from jax.experimental import pallas as pl
from jax.experimental.pallas import tpu as pltpu
from jax.experimental.pallas import tpu_sc as plsc

N = 10000
E = 320000
IN = 128
OUT = 128
NT = 25
B = 16

NC = 2
NS = 16
NW = NC * NS
EW = E // NW
CH = 80
NCH = EW // CH
RCH = 80
NRCH = N // RCH

EB1 = 512
EB2 = 2000
EB3 = 512
NB = 1000


def _sc_mesh():
    return plsc.VectorSubcoreMesh(core_axis_name="c", subcore_axis_name="s")


def _group(d):
    g = 5 if d > 16 else 25
    return g, g * CH, NCH // g


def _gather_rows(table, idx2, d):
    G, OCH, NG = _group(d)

    @functools.partial(
        pl.kernel,
        out_type=jax.ShapeDtypeStruct((E, d), jnp.float32),
        mesh=_sc_mesh(),
        compiler_params=pltpu.CompilerParams(use_tc_tiling_on_sc=False),
        scratch_types=[
            pltpu.VMEM((NCH, CH), jnp.int32),
            pltpu.VMEM((OCH, d), jnp.float32),
            pltpu.SemaphoreType.DMA,
        ],
    )
    def k(table_hbm, idx_hbm, out_hbm, idx_v, rows_v, sem):
        c = lax.axis_index("c")
        s = lax.axis_index("s")
        w = s * NC + c
        pltpu.sync_copy(idx_hbm.at[pl.ds(w * NCH, NCH)], idx_v)
        base = w * EW

        def body(g, carry):
            descs = []
            for j in range(G):
                descs.append(pltpu.async_copy(
                    table_hbm.at[idx_v.at[g * G + j]],
                    rows_v.at[pl.ds(j * CH, CH)], sem))
            for dsc in descs:
                dsc.wait()
            pltpu.sync_copy(rows_v, out_hbm.at[pl.ds(base + g * OCH, OCH)])
            return carry

        lax.fori_loop(0, NG, body, 0)

    return k(table, idx2)


def _scatter_add(msg, dst2, d):
    zeros = jnp.zeros((N, d), jnp.float32)
    G, OCH, NG = _group(d) if d <= 16 else (1, CH, NCH)

    @functools.partial(
        pl.kernel,
        out_type=jax.ShapeDtypeStruct((2, N, d), jnp.float32),
        mesh=_sc_mesh(),
        compiler_params=pltpu.CompilerParams(use_tc_tiling_on_sc=False),
        scratch_types=[
            pltpu.VMEM((NCH, CH), jnp.int32),
            pltpu.VMEM((OCH, d), jnp.float32),
            pltpu.VMEM_SHARED((N, d), jnp.float32),
            pltpu.SemaphoreType.DMA,
        ],
    )
    def k(msg_hbm, dst_hbm, zeros_hbm, out_hbm, idx_v, msg_v, accum, sem):
        c = lax.axis_index("c")
        s = lax.axis_index("s")
        w = s * NC + c
        pltpu.sync_copy(dst_hbm.at[pl.ds(w * NCH, NCH)], idx_v)

        def zbody(j, carry):
            r = (s + j * NS) * RCH
            pltpu.sync_copy(zeros_hbm.at[pl.ds(r, RCH)], accum.at[pl.ds(r, RCH)])
            return carry

        nj = (NRCH - s + NS - 1) // NS
        lax.fori_loop(0, nj, zbody, 0)
        plsc.subcore_barrier()

        base = w * EW

        def body(g, carry):
            pltpu.sync_copy(msg_hbm.at[pl.ds(base + g * OCH, OCH)], msg_v)
            descs = []
            for j in range(G):
                descs.append(pltpu.async_copy(
                    msg_v.at[pl.ds(j * CH, CH)],
                    accum.at[idx_v.at[g * G + j]], sem, add=True))
            for dsc in descs:
                dsc.wait()
            return carry

        lax.fori_loop(0, NG, body, 0)
        plsc.subcore_barrier()

        def obody(j, carry):
            r = (s + j * NS) * RCH
            pltpu.sync_copy(accum.at[pl.ds(r, RCH)], out_hbm.at[c, pl.ds(r, RCH)])
            return carry

        lax.fori_loop(0, nj, obody, 0)

    return k(msg, dst2, zeros)


BF = jnp.bfloat16


def _edge_w(ea, t_ref, eb, eo):
    et = ea[:, 0:1].astype(jnp.int32)
    oh = (et == lax.broadcasted_iota(jnp.int32, (eb, NT), 1)).astype(BF)
    f0 = ea[:, 1:2].astype(BF)
    f1 = ea[:, 2:3].astype(BF)
    m = jnp.concatenate([oh * f0, oh * f1, oh, f0, f1], axis=1)
    z = jnp.dot(m, t_ref[...], preferred_element_type=jnp.float32)
    return jnp.maximum(z, 0.0).astype(BF)


def _l1_body(ea_ref, xs_ref, t_ref, s_ref, out_ref):
    w = _edge_w(ea_ref[...], t_ref, EB1, 1280)
    xs = xs_ref[...].astype(BF)
    xs_bc = jnp.concatenate([xs] * 10, axis=1)
    out_ref[...] = jnp.dot(xs_bc * w, s_ref[...],
                           preferred_element_type=jnp.float32)
    out_ref[:, 10:11] = jnp.ones((EB1, 1), jnp.float32)


def _l2_body(ea_ref, hs_ref, t_ref, tb_ref, s_ref, out_ref):
    w = _edge_w(ea_ref[...], t_ref, EB2, 100)
    hs_bc = jnp.dot(hs_ref[...][:, 0:10].astype(BF), tb_ref[...],
                    preferred_element_type=jnp.float32).astype(BF)
    out_ref[...] = jnp.dot(hs_bc * w, s_ref[...],
                           preferred_element_type=jnp.float32)
    out_ref[:, 10:11] = jnp.ones((EB2, 1), jnp.float32)


def _l3_body(ea_ref, hs_ref, t_ref, tb_ref, s_ref, out_ref):
    w = _edge_w(ea_ref[...], t_ref, EB3, 1280)
    hs_bc = jnp.dot(hs_ref[...][:, 0:10].astype(BF), tb_ref[...],
                    preferred_element_type=jnp.float32).astype(BF)
    out_ref[...] = jnp.dot(hs_bc * w, s_ref[...],
                           preferred_element_type=jnp.float32)


def _edge_call(body, eb, d_out, ea, xs, *mats):
    blk = lambda shape: pl.BlockSpec(shape, lambda i: (0,) * len(shape))
    return pl.pallas_call(
        body,
        grid=(E // eb,),
        in_specs=[
            pl.BlockSpec((eb, 3), lambda i: (i, 0)),
            pl.BlockSpec((eb, xs.shape[1]), lambda i: (i, 0)),
        ] + [blk(m.shape) for m in mats],
        out_specs=pl.BlockSpec((eb, d_out), lambda i: (i, 0)),
        out_shape=jax.ShapeDtypeStruct((E, d_out), jnp.float32),
    )(ea, xs, *mats)


def _post_body(s_ref, xin_ref, root_ref, bias_ref, out_ref, *, ic):
    s = s_ref[0] + s_ref[1]
    cnt = jnp.maximum(s[:, 10:11], 1.0)
    agg = s[:, 0:10] / cnt
    xin = xin_ref[...][:, 0:ic]
    z = agg + jnp.dot(xin, root_ref[...],
                      preferred_element_type=jnp.float32) + bias_ref[...]
    out_ref[:, 0:10] = jnp.maximum(z, 0.0)
    out_ref[:, 10:16] = jnp.zeros((NB, 6), jnp.float32)


def _post_call(s, xin, root, bias, ic):
    return pl.pallas_call(
        functools.partial(_post_body, ic=ic),
        grid=(N // NB,),
        in_specs=[
            pl.BlockSpec((2, NB, 16), lambda i: (0, i, 0)),
            pl.BlockSpec((NB, xin.shape[1]), lambda i: (i, 0)),
            pl.BlockSpec(root.shape, lambda i: (0, 0)),
            pl.BlockSpec(bias.shape, lambda i: (0, 0)),
        ],
        out_specs=pl.BlockSpec((NB, 16), lambda i: (i, 0)),
        out_shape=jax.ShapeDtypeStruct((N, 16), jnp.float32),
    )(s, xin, root, bias)


def _post3_body(s3_ref, s1_ref, h2_ref, root_ref, bias_ref, idx_ref, out_ref,
                acc_ref, cnt_ref):
    i = pl.program_id(0)

    @pl.when(i == 0)
    def _():
        acc_ref[...] = jnp.zeros((B, 128), jnp.float32)
        cnt_ref[...] = jnp.zeros((B, 128), jnp.float32)

    s3 = s3_ref[0] + s3_ref[1]
    s1 = s1_ref[0] + s1_ref[1]
    cnt = jnp.maximum(s1[:, 10:11], 1.0)
    h3 = jnp.maximum(
        s3 / cnt + jnp.dot(h2_ref[...][:, 0:10], root_ref[...],
                           preferred_element_type=jnp.float32) + bias_ref[...],
        0.0)
    idx = idx_ref[0]
    oh = (lax.broadcasted_iota(jnp.int32, (B, NB), 0) == idx).astype(
        jnp.float32)
    acc_ref[...] += jnp.dot(oh, h3, preferred_element_type=jnp.float32)
    cnt_ref[...] += jnp.sum(oh, axis=1, keepdims=True)
    out_ref[...] = acc_ref[...] / jnp.maximum(cnt_ref[...], 1.0)


def _post3_call(s3, s1, h2, root, bias, idx3):
    return pl.pallas_call(
        _post3_body,
        grid=(N // NB,),
        in_specs=[
            pl.BlockSpec((2, NB, 128), lambda i: (0, i, 0)),
            pl.BlockSpec((2, NB, 16), lambda i: (0, i, 0)),
            pl.BlockSpec((NB, 16), lambda i: (i, 0)),
            pl.BlockSpec((10, 128), lambda i: (0, 0)),
            pl.BlockSpec((1, 128), lambda i: (0, 0)),
            pl.BlockSpec((1, 1, NB), lambda i: (i, 0, 0)),
        ],
        out_specs=pl.BlockSpec((B, 128), lambda i: (0, 0)),
        out_shape=jax.ShapeDtypeStruct((B, 128), jnp.float32),
        scratch_shapes=[
            pltpu.VMEM((B, 128), jnp.float32),
            pltpu.VMEM((B, 128), jnp.float32),
        ],
    )(s3, s1, h2, root, bias, idx3)


def _perm(a, i, o):
    lead = a.shape[:-1]
    return a.reshape(lead + (i, o)).swapaxes(-1, -2).reshape(lead + (i * o,))


def kernel(x, edge_index, edge_attr, cell_type, batch,
           emb1, Wh1, bh1, Wg1, bg1, root1, bias1,
           emb2, Wh2, bh2, Wg2, bg2, root2, bias2,
           emb3, Wh3, bh3, Wg3, bg3, root3, bias3):
    src2 = edge_index[0].reshape(E // CH, CH)
    dst2 = edge_index[1].reshape(E // CH, CH)
    ea = edge_attr

    def edge_t(emb_, wh_, bh_, wg_, bg_):
        return jnp.concatenate([
            emb_ * wh_[0:1], emb_ * wh_[1:2],
            emb_ * bh_[None, :] + bg_[None, :],
            wg_[0:1], wg_[1:2]], axis=0)

    t1 = edge_t(_perm(emb1, IN, 10), _perm(Wh1, IN, 10), _perm(bh1, IN, 10),
                _perm(Wg1, IN, 10), _perm(bg1, IN, 10))
    t2 = edge_t(_perm(emb2, 10, 10), _perm(Wh2, 10, 10), _perm(bh2, 10, 10),
                _perm(Wg2, 10, 10), _perm(bg2, 10, 10))
    t3 = edge_t(emb3, Wh3, bh3, Wg3, bg3)

    bf = jnp.bfloat16
    t1 = t1.astype(bf)
    t2 = t2.astype(bf)
    t3 = t3.astype(bf)
    seg1 = jnp.arange(1280) // 128
    s1m = (seg1[:, None] == jnp.arange(16)[None, :]).astype(bf)
    seg2 = jnp.arange(100) // 10
    s2m = (seg2[:, None] == jnp.arange(16)[None, :]).astype(bf)
    tb2 = (jnp.arange(10)[:, None] == (jnp.arange(100) % 10)[None, :]).astype(
        bf)
    tb3 = (jnp.arange(10)[:, None] == seg1[None, :]).astype(bf)
    s3m = (seg1[:, None] * 0 + jnp.arange(1280)[:, None] % 128
           == jnp.arange(128)[None, :]).astype(bf)

    bias1r = bias1[None, :]
    bias2r = bias2[None, :]
    bias3r = bias3[None, :]

    idx3 = ((cell_type == 1).astype(jnp.int32) * (batch + 1) - 1).reshape(
        N // NB, 1, NB)

    xs = _gather_rows(x, src2, 128)
    msg1 = _edge_call(_l1_body, EB1, 16, ea, xs, t1, s1m)
    s1 = _scatter_add(msg1, dst2, 16)
    h1 = _post_call(s1, x, root1, bias1r, ic=128)

    hs2 = _gather_rows(h1, src2, 16)
    msg2 = _edge_call(_l2_body, EB2, 16, ea, hs2, t2, tb2, s2m)
    s2 = _scatter_add(msg2, dst2, 16)
    h2 = _post_call(s2, h1, root2, bias2r, ic=10)

    hs3 = _gather_rows(h2, src2, 16)
    msg3 = _edge_call(_l3_body, EB3, 128, ea, hs3, t3, tb3, s3m)
    s3 = _scatter_add(msg3, dst2, 128)
    return _post3_call(s3, s1, h2, root3, bias3r, idx3)


# speedup vs baseline: 1.1448x; 1.1448x over previous
import functools

import jax
import jax.numpy as jnp
from jax import lax
from jax.experimental import pallas as pl
from jax.experimental.pallas import tpu as pltpu
from jax.experimental.pallas import tpu_sc as plsc

N = 10000
E = 320000
IN = 128
OUT = 128
NT = 25
B = 16

NC = 2
NS = 16
NW = NC * NS
EW = E // NW
CH = 80
NCH = EW // CH
RCH = 80
NRCH = N // RCH

EB1 = 512
EB2 = 2000
EB3 = 512
NB = 1000


def _sc_mesh():
    return plsc.VectorSubcoreMesh(core_axis_name="c", subcore_axis_name="s")


def _group(d):
    g = 5 if d > 16 else 25
    return g, g * CH, NCH // g


def _gather_rows(table, idx2, d):
    G, OCH, NG = _group(d)

    @functools.partial(
        pl.kernel,
        out_type=jax.ShapeDtypeStruct((E, d), jnp.float32),
        mesh=_sc_mesh(),
        compiler_params=pltpu.CompilerParams(use_tc_tiling_on_sc=False),
        scratch_types=[
            pltpu.VMEM((NCH, CH), jnp.int32),
            pltpu.VMEM((OCH, d), jnp.float32),
            pltpu.SemaphoreType.DMA,
        ],
    )
    def k(table_hbm, idx_hbm, out_hbm, idx_v, rows_v, sem):
        c = lax.axis_index("c")
        s = lax.axis_index("s")
        w = s * NC + c
        pltpu.sync_copy(idx_hbm.at[pl.ds(w * NCH, NCH)], idx_v)
        base = w * EW

        def body(g, carry):
            descs = []
            for j in range(G):
                descs.append(pltpu.async_copy(
                    table_hbm.at[idx_v.at[g * G + j]],
                    rows_v.at[pl.ds(j * CH, CH)], sem))
            for dsc in descs:
                dsc.wait()
            pltpu.sync_copy(rows_v, out_hbm.at[pl.ds(base + g * OCH, OCH)])
            return carry

        lax.fori_loop(0, NG, body, 0)

    return k(table, idx2)


def _scatter_add(msg, dst2, d):
    zeros = jnp.zeros((N, d), jnp.float32)
    G, OCH, NG = _group(d) if d <= 16 else (1, CH, NCH)

    @functools.partial(
        pl.kernel,
        out_type=jax.ShapeDtypeStruct((2, N, d), jnp.float32),
        mesh=_sc_mesh(),
        compiler_params=pltpu.CompilerParams(use_tc_tiling_on_sc=False),
        scratch_types=[
            pltpu.VMEM((NCH, CH), jnp.int32),
            pltpu.VMEM((OCH, d), jnp.float32),
            pltpu.VMEM_SHARED((N, d), jnp.float32),
            pltpu.SemaphoreType.DMA,
        ],
    )
    def k(msg_hbm, dst_hbm, zeros_hbm, out_hbm, idx_v, msg_v, accum, sem):
        c = lax.axis_index("c")
        s = lax.axis_index("s")
        w = s * NC + c
        pltpu.sync_copy(dst_hbm.at[pl.ds(w * NCH, NCH)], idx_v)

        def zbody(j, carry):
            r = (s + j * NS) * RCH
            pltpu.sync_copy(zeros_hbm.at[pl.ds(r, RCH)], accum.at[pl.ds(r, RCH)])
            return carry

        nj = (NRCH - s + NS - 1) // NS
        lax.fori_loop(0, nj, zbody, 0)
        plsc.subcore_barrier()

        base = w * EW

        def body(g, carry):
            pltpu.sync_copy(msg_hbm.at[pl.ds(base + g * OCH, OCH)], msg_v)
            descs = []
            for j in range(G):
                descs.append(pltpu.async_copy(
                    msg_v.at[pl.ds(j * CH, CH)],
                    accum.at[idx_v.at[g * G + j]], sem, add=True))
            for dsc in descs:
                dsc.wait()
            return carry

        lax.fori_loop(0, NG, body, 0)
        plsc.subcore_barrier()

        def obody(j, carry):
            r = (s + j * NS) * RCH
            pltpu.sync_copy(accum.at[pl.ds(r, RCH)], out_hbm.at[c, pl.ds(r, RCH)])
            return carry

        lax.fori_loop(0, nj, obody, 0)

    return k(msg, dst2, zeros)


BF = jnp.bfloat16


def _edge_w(ea, t_ref, eb, eo):
    et = ea[:, 0:1].astype(jnp.int32)
    oh = (et == lax.broadcasted_iota(jnp.int32, (eb, NT), 1)).astype(BF)
    f0 = ea[:, 1:2].astype(BF)
    f1 = ea[:, 2:3].astype(BF)
    m = jnp.concatenate([oh * f0, oh * f1, oh, f0, f1], axis=1)
    z = jnp.dot(m, t_ref[...], preferred_element_type=jnp.float32)
    return jnp.maximum(z, 0.0).astype(BF)


def _l1_body(ea_ref, xs_ref, t_ref, s_ref, out_ref):
    w = _edge_w(ea_ref[...], t_ref, EB1, 1280)
    xs = xs_ref[...].astype(BF)
    xs_bc = jnp.concatenate([xs] * 10, axis=1)
    out_ref[...] = jnp.dot(xs_bc * w, s_ref[...],
                           preferred_element_type=jnp.float32)
    out_ref[:, 10:11] = jnp.ones((EB1, 1), jnp.float32)


def _l2_body(ea_ref, hs_ref, t_ref, tb_ref, s_ref, out_ref):
    w = _edge_w(ea_ref[...], t_ref, EB2, 100)
    hs_bc = jnp.dot(hs_ref[...][:, 0:10].astype(BF), tb_ref[...],
                    preferred_element_type=jnp.float32).astype(BF)
    out_ref[...] = jnp.dot(hs_bc * w, s_ref[...],
                           preferred_element_type=jnp.float32)
    out_ref[:, 10:11] = jnp.ones((EB2, 1), jnp.float32)


def _l3_body(ea_ref, hs_ref, t_ref, tb_ref, out_ref):
    w = _edge_w(ea_ref[...], t_ref, EB3, 1280)
    hs_bc = jnp.dot(hs_ref[...][:, 0:10].astype(BF), tb_ref[...],
                    preferred_element_type=jnp.float32).astype(BF)
    prod = hs_bc * w
    acc = prod[:, 0:128]
    for i in range(1, 10):
        acc = acc + prod[:, i * 128:(i + 1) * 128]
    out_ref[...] = acc.astype(jnp.float32)


def _edge_call(body, eb, d_out, ea, xs, *mats):
    blk = lambda shape: pl.BlockSpec(shape, lambda i: (0,) * len(shape))
    return pl.pallas_call(
        body,
        grid=(E // eb,),
        in_specs=[
            pl.BlockSpec((eb, 3), lambda i: (i, 0)),
            pl.BlockSpec((eb, xs.shape[1]), lambda i: (i, 0)),
        ] + [blk(m.shape) for m in mats],
        out_specs=pl.BlockSpec((eb, d_out), lambda i: (i, 0)),
        out_shape=jax.ShapeDtypeStruct((E, d_out), jnp.float32),
    )(ea, xs, *mats)


def _post_body(s_ref, xin_ref, root_ref, bias_ref, out_ref, *, ic):
    s = s_ref[0] + s_ref[1]
    cnt = jnp.maximum(s[:, 10:11], 1.0)
    agg = s[:, 0:10] / cnt
    xin = xin_ref[...][:, 0:ic]
    z = agg + jnp.dot(xin, root_ref[...],
                      preferred_element_type=jnp.float32) + bias_ref[...]
    out_ref[:, 0:10] = jnp.maximum(z, 0.0)
    out_ref[:, 10:16] = jnp.zeros((NB, 6), jnp.float32)


def _post_call(s, xin, root, bias, ic):
    return pl.pallas_call(
        functools.partial(_post_body, ic=ic),
        grid=(N // NB,),
        in_specs=[
            pl.BlockSpec((2, NB, 16), lambda i: (0, i, 0)),
            pl.BlockSpec((NB, xin.shape[1]), lambda i: (i, 0)),
            pl.BlockSpec(root.shape, lambda i: (0, 0)),
            pl.BlockSpec(bias.shape, lambda i: (0, 0)),
        ],
        out_specs=pl.BlockSpec((NB, 16), lambda i: (i, 0)),
        out_shape=jax.ShapeDtypeStruct((N, 16), jnp.float32),
    )(s, xin, root, bias)


def _post3_body(s3_ref, s1_ref, h2_ref, root_ref, bias_ref, idx_ref, out_ref,
                acc_ref, cnt_ref):
    i = pl.program_id(0)

    @pl.when(i == 0)
    def _():
        acc_ref[...] = jnp.zeros((B, 128), jnp.float32)
        cnt_ref[...] = jnp.zeros((B, 128), jnp.float32)

    s3 = s3_ref[0] + s3_ref[1]
    s1 = s1_ref[0] + s1_ref[1]
    cnt = jnp.maximum(s1[:, 10:11], 1.0)
    h3 = jnp.maximum(
        s3 / cnt + jnp.dot(h2_ref[...][:, 0:10], root_ref[...],
                           preferred_element_type=jnp.float32) + bias_ref[...],
        0.0)
    idx = idx_ref[0]
    oh = (lax.broadcasted_iota(jnp.int32, (B, NB), 0) == idx).astype(
        jnp.float32)
    acc_ref[...] += jnp.dot(oh, h3, preferred_element_type=jnp.float32)
    cnt_ref[...] += jnp.sum(oh, axis=1, keepdims=True)
    out_ref[...] = acc_ref[...] / jnp.maximum(cnt_ref[...], 1.0)


def _post3_call(s3, s1, h2, root, bias, idx3):
    return pl.pallas_call(
        _post3_body,
        grid=(N // NB,),
        in_specs=[
            pl.BlockSpec((2, NB, 128), lambda i: (0, i, 0)),
            pl.BlockSpec((2, NB, 16), lambda i: (0, i, 0)),
            pl.BlockSpec((NB, 16), lambda i: (i, 0)),
            pl.BlockSpec((10, 128), lambda i: (0, 0)),
            pl.BlockSpec((1, 128), lambda i: (0, 0)),
            pl.BlockSpec((1, 1, NB), lambda i: (i, 0, 0)),
        ],
        out_specs=pl.BlockSpec((B, 128), lambda i: (0, 0)),
        out_shape=jax.ShapeDtypeStruct((B, 128), jnp.float32),
        scratch_shapes=[
            pltpu.VMEM((B, 128), jnp.float32),
            pltpu.VMEM((B, 128), jnp.float32),
        ],
    )(s3, s1, h2, root, bias, idx3)


def _perm(a, i, o):
    lead = a.shape[:-1]
    return a.reshape(lead + (i, o)).swapaxes(-1, -2).reshape(lead + (i * o,))


def kernel(x, edge_index, edge_attr, cell_type, batch,
           emb1, Wh1, bh1, Wg1, bg1, root1, bias1,
           emb2, Wh2, bh2, Wg2, bg2, root2, bias2,
           emb3, Wh3, bh3, Wg3, bg3, root3, bias3):
    src2 = edge_index[0].reshape(E // CH, CH)
    dst2 = edge_index[1].reshape(E // CH, CH)
    ea = edge_attr

    def edge_t(emb_, wh_, bh_, wg_, bg_):
        return jnp.concatenate([
            emb_ * wh_[0:1], emb_ * wh_[1:2],
            emb_ * bh_[None, :] + bg_[None, :],
            wg_[0:1], wg_[1:2]], axis=0)

    t1 = edge_t(_perm(emb1, IN, 10), _perm(Wh1, IN, 10), _perm(bh1, IN, 10),
                _perm(Wg1, IN, 10), _perm(bg1, IN, 10))
    t2 = edge_t(_perm(emb2, 10, 10), _perm(Wh2, 10, 10), _perm(bh2, 10, 10),
                _perm(Wg2, 10, 10), _perm(bg2, 10, 10))
    t3 = edge_t(emb3, Wh3, bh3, Wg3, bg3)

    bf = jnp.bfloat16
    t1 = t1.astype(bf)
    t2 = t2.astype(bf)
    t3 = t3.astype(bf)
    seg1 = jnp.arange(1280) // 128
    s1m = (seg1[:, None] == jnp.arange(16)[None, :]).astype(bf)
    seg2 = jnp.arange(100) // 10
    s2m = (seg2[:, None] == jnp.arange(16)[None, :]).astype(bf)
    tb2 = (jnp.arange(10)[:, None] == (jnp.arange(100) % 10)[None, :]).astype(
        bf)
    tb3 = (jnp.arange(10)[:, None] == seg1[None, :]).astype(bf)

    bias1r = bias1[None, :]
    bias2r = bias2[None, :]
    bias3r = bias3[None, :]

    idx3 = ((cell_type == 1).astype(jnp.int32) * (batch + 1) - 1).reshape(
        N // NB, 1, NB)

    xs = _gather_rows(x, src2, 128)
    msg1 = _edge_call(_l1_body, EB1, 16, ea, xs, t1, s1m)
    s1 = _scatter_add(msg1, dst2, 16)
    h1 = _post_call(s1, x, root1, bias1r, ic=128)

    hs2 = _gather_rows(h1, src2, 16)
    msg2 = _edge_call(_l2_body, EB2, 16, ea, hs2, t2, tb2, s2m)
    s2 = _scatter_add(msg2, dst2, 16)
    h2 = _post_call(s2, h1, root2, bias2r, ic=10)

    hs3 = _gather_rows(h2, src2, 16)
    msg3 = _edge_call(_l3_body, EB3, 128, ea, hs3, t3, tb3)
    s3 = _scatter_add(msg3, dst2, 128)
    return _post3_call(s3, s1, h2, root3, bias3r, idx3)


# speedup vs baseline: 1.2370x; 1.0806x over previous
import functools

import jax
import jax.numpy as jnp
from jax import lax
from jax.experimental import pallas as pl
from jax.experimental.pallas import tpu as pltpu
from jax.experimental.pallas import tpu_sc as plsc

N = 10000
E = 320000
IN = 128
OUT = 128
NT = 25
B = 16

NC = 2
NS = 16
NW = NC * NS
EW = E // NW
CH = 80
NCH = EW // CH
RCH = 80
NRCH = N // RCH

EB1 = 1000
EB2 = 2000
EB3 = 1000
NB = 1000


def _sc_mesh():
    return plsc.VectorSubcoreMesh(core_axis_name="c", subcore_axis_name="s")


def _group(d):
    g = 5 if d > 16 else 25
    return g, g * CH, NCH // g


def _gather_rows(table, idx2, d):
    G, OCH, NG = _group(d)

    @functools.partial(
        pl.kernel,
        out_type=jax.ShapeDtypeStruct((E, d), jnp.float32),
        mesh=_sc_mesh(),
        compiler_params=pltpu.CompilerParams(use_tc_tiling_on_sc=False),
        scratch_types=[
            pltpu.VMEM((NCH, CH), jnp.int32),
            pltpu.VMEM((OCH, d), jnp.float32),
            pltpu.SemaphoreType.DMA,
        ],
    )
    def k(table_hbm, idx_hbm, out_hbm, idx_v, rows_v, sem):
        c = lax.axis_index("c")
        s = lax.axis_index("s")
        w = s * NC + c
        pltpu.sync_copy(idx_hbm.at[pl.ds(w * NCH, NCH)], idx_v)
        base = w * EW

        def body(g, carry):
            descs = []
            for j in range(G):
                descs.append(pltpu.async_copy(
                    table_hbm.at[idx_v.at[g * G + j]],
                    rows_v.at[pl.ds(j * CH, CH)], sem))
            for dsc in descs:
                dsc.wait()
            pltpu.sync_copy(rows_v, out_hbm.at[pl.ds(base + g * OCH, OCH)])
            return carry

        lax.fori_loop(0, NG, body, 0)

    return k(table, idx2)


def _scatter_add(msg, dst2, d):
    zeros = jnp.zeros((N, d), jnp.float32)
    G, OCH, NG = _group(d) if d <= 16 else (1, CH, NCH)

    @functools.partial(
        pl.kernel,
        out_type=jax.ShapeDtypeStruct((2, N, d), jnp.float32),
        mesh=_sc_mesh(),
        compiler_params=pltpu.CompilerParams(use_tc_tiling_on_sc=False),
        scratch_types=[
            pltpu.VMEM((NCH, CH), jnp.int32),
            pltpu.VMEM((OCH, d), jnp.float32),
            pltpu.VMEM_SHARED((N, d), jnp.float32),
            pltpu.SemaphoreType.DMA,
        ],
    )
    def k(msg_hbm, dst_hbm, zeros_hbm, out_hbm, idx_v, msg_v, accum, sem):
        c = lax.axis_index("c")
        s = lax.axis_index("s")
        w = s * NC + c
        pltpu.sync_copy(dst_hbm.at[pl.ds(w * NCH, NCH)], idx_v)

        def zbody(j, carry):
            r = (s + j * NS) * RCH
            pltpu.sync_copy(zeros_hbm.at[pl.ds(r, RCH)], accum.at[pl.ds(r, RCH)])
            return carry

        nj = (NRCH - s + NS - 1) // NS
        lax.fori_loop(0, nj, zbody, 0)
        plsc.subcore_barrier()

        base = w * EW

        def body(g, carry):
            pltpu.sync_copy(msg_hbm.at[pl.ds(base + g * OCH, OCH)], msg_v)
            descs = []
            for j in range(G):
                descs.append(pltpu.async_copy(
                    msg_v.at[pl.ds(j * CH, CH)],
                    accum.at[idx_v.at[g * G + j]], sem, add=True))
            for dsc in descs:
                dsc.wait()
            return carry

        lax.fori_loop(0, NG, body, 0)
        plsc.subcore_barrier()

        def obody(j, carry):
            r = (s + j * NS) * RCH
            pltpu.sync_copy(accum.at[pl.ds(r, RCH)], out_hbm.at[c, pl.ds(r, RCH)])
            return carry

        lax.fori_loop(0, nj, obody, 0)

    return k(msg, dst2, zeros)


BF = jnp.bfloat16


def _edge_w(ea, t_ref, eb, eo):
    et = ea[:, 0:1].astype(jnp.int32)
    oh = (et == lax.broadcasted_iota(jnp.int32, (eb, NT), 1)).astype(BF)
    f0 = ea[:, 1:2].astype(BF)
    f1 = ea[:, 2:3].astype(BF)
    m = jnp.concatenate([oh * f0, oh * f1, oh, f0, f1], axis=1)
    z = jnp.dot(m, t_ref[...], preferred_element_type=jnp.float32)
    return jnp.maximum(z, 0.0).astype(BF)


def _l1_body(ea_ref, xs_ref, t_ref, s_ref, out_ref):
    w = _edge_w(ea_ref[...], t_ref, EB1, 1280)
    xs = xs_ref[...].astype(BF)
    xs_bc = jnp.concatenate([xs] * 10, axis=1)
    out_ref[...] = jnp.dot(xs_bc * w, s_ref[...],
                           preferred_element_type=jnp.float32)
    out_ref[:, 10:11] = jnp.ones((EB1, 1), jnp.float32)


def _l2_body(ea_ref, hs_ref, t_ref, tb_ref, s_ref, out_ref):
    w = _edge_w(ea_ref[...], t_ref, EB2, 100)
    hs_bc = jnp.dot(hs_ref[...][:, 0:10].astype(BF), tb_ref[...],
                    preferred_element_type=jnp.float32).astype(BF)
    out_ref[...] = jnp.dot(hs_bc * w, s_ref[...],
                           preferred_element_type=jnp.float32)
    out_ref[:, 10:11] = jnp.ones((EB2, 1), jnp.float32)


def _l3_body(ea_ref, hs_ref, t_ref, tb_ref, out_ref):
    w = _edge_w(ea_ref[...], t_ref, EB3, 1280)
    hs_bc = jnp.dot(hs_ref[...][:, 0:10].astype(BF), tb_ref[...],
                    preferred_element_type=jnp.float32).astype(BF)
    prod = hs_bc * w
    acc = prod[:, 0:128]
    for i in range(1, 10):
        acc = acc + prod[:, i * 128:(i + 1) * 128]
    out_ref[...] = acc.astype(jnp.float32)


def _edge_call(body, eb, d_out, ea, xs, *mats):
    blk = lambda shape: pl.BlockSpec(shape, lambda i: (0,) * len(shape))
    return pl.pallas_call(
        body,
        grid=(E // eb,),
        in_specs=[
            pl.BlockSpec((eb, 3), lambda i: (i, 0)),
            pl.BlockSpec((eb, xs.shape[1]), lambda i: (i, 0)),
        ] + [blk(m.shape) for m in mats],
        out_specs=pl.BlockSpec((eb, d_out), lambda i: (i, 0)),
        out_shape=jax.ShapeDtypeStruct((E, d_out), jnp.float32),
    )(ea, xs, *mats)


def _post_body(s_ref, xin_ref, root_ref, bias_ref, out_ref, *, ic):
    s = s_ref[0] + s_ref[1]
    cnt = jnp.maximum(s[:, 10:11], 1.0)
    agg = s[:, 0:10] / cnt
    xin = xin_ref[...][:, 0:ic]
    z = agg + jnp.dot(xin, root_ref[...],
                      preferred_element_type=jnp.float32) + bias_ref[...]
    out_ref[:, 0:10] = jnp.maximum(z, 0.0)
    out_ref[:, 10:16] = jnp.zeros((NB, 6), jnp.float32)


def _post_call(s, xin, root, bias, ic):
    return pl.pallas_call(
        functools.partial(_post_body, ic=ic),
        grid=(N // NB,),
        in_specs=[
            pl.BlockSpec((2, NB, 16), lambda i: (0, i, 0)),
            pl.BlockSpec((NB, xin.shape[1]), lambda i: (i, 0)),
            pl.BlockSpec(root.shape, lambda i: (0, 0)),
            pl.BlockSpec(bias.shape, lambda i: (0, 0)),
        ],
        out_specs=pl.BlockSpec((NB, 16), lambda i: (i, 0)),
        out_shape=jax.ShapeDtypeStruct((N, 16), jnp.float32),
    )(s, xin, root, bias)


def _post3_body(s3_ref, s1_ref, h2_ref, root_ref, bias_ref, idx_ref, out_ref,
                acc_ref, cnt_ref):
    i = pl.program_id(0)

    @pl.when(i == 0)
    def _():
        acc_ref[...] = jnp.zeros((B, 128), jnp.float32)
        cnt_ref[...] = jnp.zeros((B, 128), jnp.float32)

    s3 = s3_ref[0] + s3_ref[1]
    s1 = s1_ref[0] + s1_ref[1]
    cnt = jnp.maximum(s1[:, 10:11], 1.0)
    h3 = jnp.maximum(
        s3 / cnt + jnp.dot(h2_ref[...][:, 0:10], root_ref[...],
                           preferred_element_type=jnp.float32) + bias_ref[...],
        0.0)
    idx = idx_ref[0]
    oh = (lax.broadcasted_iota(jnp.int32, (B, NB), 0) == idx).astype(
        jnp.float32)
    acc_ref[...] += jnp.dot(oh, h3, preferred_element_type=jnp.float32)
    cnt_ref[...] += jnp.sum(oh, axis=1, keepdims=True)
    out_ref[...] = acc_ref[...] / jnp.maximum(cnt_ref[...], 1.0)


def _post3_call(s3, s1, h2, root, bias, idx3):
    return pl.pallas_call(
        _post3_body,
        grid=(N // NB,),
        in_specs=[
            pl.BlockSpec((2, NB, 128), lambda i: (0, i, 0)),
            pl.BlockSpec((2, NB, 16), lambda i: (0, i, 0)),
            pl.BlockSpec((NB, 16), lambda i: (i, 0)),
            pl.BlockSpec((10, 128), lambda i: (0, 0)),
            pl.BlockSpec((1, 128), lambda i: (0, 0)),
            pl.BlockSpec((1, 1, NB), lambda i: (i, 0, 0)),
        ],
        out_specs=pl.BlockSpec((B, 128), lambda i: (0, 0)),
        out_shape=jax.ShapeDtypeStruct((B, 128), jnp.float32),
        scratch_shapes=[
            pltpu.VMEM((B, 128), jnp.float32),
            pltpu.VMEM((B, 128), jnp.float32),
        ],
    )(s3, s1, h2, root, bias, idx3)


def _perm(a, i, o):
    lead = a.shape[:-1]
    return a.reshape(lead + (i, o)).swapaxes(-1, -2).reshape(lead + (i * o,))


def kernel(x, edge_index, edge_attr, cell_type, batch,
           emb1, Wh1, bh1, Wg1, bg1, root1, bias1,
           emb2, Wh2, bh2, Wg2, bg2, root2, bias2,
           emb3, Wh3, bh3, Wg3, bg3, root3, bias3):
    src2 = edge_index[0].reshape(E // CH, CH)
    dst2 = edge_index[1].reshape(E // CH, CH)
    ea = edge_attr

    def edge_t(emb_, wh_, bh_, wg_, bg_):
        return jnp.concatenate([
            emb_ * wh_[0:1], emb_ * wh_[1:2],
            emb_ * bh_[None, :] + bg_[None, :],
            wg_[0:1], wg_[1:2]], axis=0)

    t1 = edge_t(_perm(emb1, IN, 10), _perm(Wh1, IN, 10), _perm(bh1, IN, 10),
                _perm(Wg1, IN, 10), _perm(bg1, IN, 10))
    t2 = edge_t(_perm(emb2, 10, 10), _perm(Wh2, 10, 10), _perm(bh2, 10, 10),
                _perm(Wg2, 10, 10), _perm(bg2, 10, 10))
    t3 = edge_t(emb3, Wh3, bh3, Wg3, bg3)

    bf = jnp.bfloat16
    t1 = t1.astype(bf)
    t2 = t2.astype(bf)
    t3 = t3.astype(bf)
    seg1 = jnp.arange(1280) // 128
    s1m = (seg1[:, None] == jnp.arange(16)[None, :]).astype(bf)
    seg2 = jnp.arange(100) // 10
    s2m = (seg2[:, None] == jnp.arange(16)[None, :]).astype(bf)
    tb2 = (jnp.arange(10)[:, None] == (jnp.arange(100) % 10)[None, :]).astype(
        bf)
    tb3 = (jnp.arange(10)[:, None] == seg1[None, :]).astype(bf)

    bias1r = bias1[None, :]
    bias2r = bias2[None, :]
    bias3r = bias3[None, :]

    idx3 = ((cell_type == 1).astype(jnp.int32) * (batch + 1) - 1).reshape(
        N // NB, 1, NB)

    xs = _gather_rows(x, src2, 128)
    msg1 = _edge_call(_l1_body, EB1, 16, ea, xs, t1, s1m)
    s1 = _scatter_add(msg1, dst2, 16)
    h1 = _post_call(s1, x, root1, bias1r, ic=128)

    hs2 = _gather_rows(h1, src2, 16)
    msg2 = _edge_call(_l2_body, EB2, 16, ea, hs2, t2, tb2, s2m)
    s2 = _scatter_add(msg2, dst2, 16)
    h2 = _post_call(s2, h1, root2, bias2r, ic=10)

    hs3 = _gather_rows(h2, src2, 16)
    msg3 = _edge_call(_l3_body, EB3, 128, ea, hs3, t3, tb3)
    s3 = _scatter_add(msg3, dst2, 128)
    return _post3_call(s3, s1, h2, root3, bias3r, idx3)


# speedup vs baseline: 1.2885x; 1.0416x over previous
import functools

import jax
import jax.numpy as jnp
from jax import lax
from jax.experimental import pallas as pl
from jax.experimental.pallas import tpu as pltpu
from jax.experimental.pallas import tpu_sc as plsc

N = 10000
E = 320000
IN = 128
OUT = 128
NT = 25
B = 16

NC = 2
NS = 16
NW = NC * NS
EW = E // NW
CH = 80
NCH = EW // CH
RCH = 80
NRCH = N // RCH

EB1 = 2000
EB2 = 2000
EB3 = 2000
NB = 1000


def _sc_mesh():
    return plsc.VectorSubcoreMesh(core_axis_name="c", subcore_axis_name="s")


def _group(d):
    g = 5 if d > 16 else 25
    return g, g * CH, NCH // g


def _gather_rows(table, idx2, d):
    G, OCH, NG = _group(d)

    @functools.partial(
        pl.kernel,
        out_type=jax.ShapeDtypeStruct((E, d), jnp.float32),
        mesh=_sc_mesh(),
        compiler_params=pltpu.CompilerParams(use_tc_tiling_on_sc=False),
        scratch_types=[
            pltpu.VMEM((NCH, CH), jnp.int32),
            pltpu.VMEM((OCH, d), jnp.float32),
            pltpu.SemaphoreType.DMA,
        ],
    )
    def k(table_hbm, idx_hbm, out_hbm, idx_v, rows_v, sem):
        c = lax.axis_index("c")
        s = lax.axis_index("s")
        w = s * NC + c
        pltpu.sync_copy(idx_hbm.at[pl.ds(w * NCH, NCH)], idx_v)
        base = w * EW

        def body(g, carry):
            descs = []
            for j in range(G):
                descs.append(pltpu.async_copy(
                    table_hbm.at[idx_v.at[g * G + j]],
                    rows_v.at[pl.ds(j * CH, CH)], sem))
            for dsc in descs:
                dsc.wait()
            pltpu.sync_copy(rows_v, out_hbm.at[pl.ds(base + g * OCH, OCH)])
            return carry

        lax.fori_loop(0, NG, body, 0)

    return k(table, idx2)


def _scatter_add(msg, dst2, d):
    zeros = jnp.zeros((N, d), jnp.float32)
    G, OCH, NG = _group(d) if d <= 16 else (1, CH, NCH)

    @functools.partial(
        pl.kernel,
        out_type=jax.ShapeDtypeStruct((2, N, d), jnp.float32),
        mesh=_sc_mesh(),
        compiler_params=pltpu.CompilerParams(use_tc_tiling_on_sc=False),
        scratch_types=[
            pltpu.VMEM((NCH, CH), jnp.int32),
            pltpu.VMEM((OCH, d), jnp.float32),
            pltpu.VMEM_SHARED((N, d), jnp.float32),
            pltpu.SemaphoreType.DMA,
        ],
    )
    def k(msg_hbm, dst_hbm, zeros_hbm, out_hbm, idx_v, msg_v, accum, sem):
        c = lax.axis_index("c")
        s = lax.axis_index("s")
        w = s * NC + c
        pltpu.sync_copy(dst_hbm.at[pl.ds(w * NCH, NCH)], idx_v)

        def zbody(j, carry):
            r = (s + j * NS) * RCH
            pltpu.sync_copy(zeros_hbm.at[pl.ds(r, RCH)], accum.at[pl.ds(r, RCH)])
            return carry

        nj = (NRCH - s + NS - 1) // NS
        lax.fori_loop(0, nj, zbody, 0)
        plsc.subcore_barrier()

        base = w * EW

        def body(g, carry):
            pltpu.sync_copy(msg_hbm.at[pl.ds(base + g * OCH, OCH)], msg_v)
            descs = []
            for j in range(G):
                descs.append(pltpu.async_copy(
                    msg_v.at[pl.ds(j * CH, CH)],
                    accum.at[idx_v.at[g * G + j]], sem, add=True))
            for dsc in descs:
                dsc.wait()
            return carry

        lax.fori_loop(0, NG, body, 0)
        plsc.subcore_barrier()

        def obody(j, carry):
            r = (s + j * NS) * RCH
            pltpu.sync_copy(accum.at[pl.ds(r, RCH)], out_hbm.at[c, pl.ds(r, RCH)])
            return carry

        lax.fori_loop(0, nj, obody, 0)

    return k(msg, dst2, zeros)


BF = jnp.bfloat16


def _edge_w(ea, t_ref, eb, eo):
    et = ea[:, 0:1].astype(jnp.int32)
    oh = (et == lax.broadcasted_iota(jnp.int32, (eb, NT), 1)).astype(BF)
    f0 = ea[:, 1:2].astype(BF)
    f1 = ea[:, 2:3].astype(BF)
    m = jnp.concatenate([oh * f0, oh * f1, oh, f0, f1], axis=1)
    z = jnp.dot(m, t_ref[...], preferred_element_type=jnp.float32)
    return jnp.maximum(z, 0.0).astype(BF)


def _l1_body(ea_ref, xs_ref, t_ref, s_ref, out_ref):
    w = _edge_w(ea_ref[...], t_ref, EB1, 1280)
    xs = xs_ref[...].astype(BF)
    xs_bc = jnp.concatenate([xs] * 10, axis=1)
    out_ref[...] = jnp.dot(xs_bc * w, s_ref[...],
                           preferred_element_type=jnp.float32)
    out_ref[:, 10:11] = jnp.ones((EB1, 1), jnp.float32)


def _l2_body(ea_ref, hs_ref, t_ref, tb_ref, s_ref, out_ref):
    w = _edge_w(ea_ref[...], t_ref, EB2, 100)
    hs_bc = jnp.dot(hs_ref[...][:, 0:10].astype(BF), tb_ref[...],
                    preferred_element_type=jnp.float32).astype(BF)
    out_ref[...] = jnp.dot(hs_bc * w, s_ref[...],
                           preferred_element_type=jnp.float32)
    out_ref[:, 10:11] = jnp.ones((EB2, 1), jnp.float32)


def _l3_body(ea_ref, hs_ref, t_ref, tb_ref, out_ref):
    w = _edge_w(ea_ref[...], t_ref, EB3, 1280)
    hs_bc = jnp.dot(hs_ref[...][:, 0:10].astype(BF), tb_ref[...],
                    preferred_element_type=jnp.float32).astype(BF)
    prod = hs_bc * w
    acc = prod[:, 0:128]
    for i in range(1, 10):
        acc = acc + prod[:, i * 128:(i + 1) * 128]
    out_ref[...] = acc.astype(jnp.float32)


def _edge_call(body, eb, d_out, ea, xs, *mats):
    blk = lambda shape: pl.BlockSpec(shape, lambda i: (0,) * len(shape))
    return pl.pallas_call(
        body,
        grid=(E // eb,),
        in_specs=[
            pl.BlockSpec((eb, 3), lambda i: (i, 0)),
            pl.BlockSpec((eb, xs.shape[1]), lambda i: (i, 0)),
        ] + [blk(m.shape) for m in mats],
        out_specs=pl.BlockSpec((eb, d_out), lambda i: (i, 0)),
        out_shape=jax.ShapeDtypeStruct((E, d_out), jnp.float32),
    )(ea, xs, *mats)


def _post_body(s_ref, xin_ref, root_ref, bias_ref, out_ref, *, ic):
    s = s_ref[0] + s_ref[1]
    cnt = jnp.maximum(s[:, 10:11], 1.0)
    agg = s[:, 0:10] / cnt
    xin = xin_ref[...][:, 0:ic]
    z = agg + jnp.dot(xin, root_ref[...],
                      preferred_element_type=jnp.float32) + bias_ref[...]
    out_ref[:, 0:10] = jnp.maximum(z, 0.0)
    out_ref[:, 10:16] = jnp.zeros((NB, 6), jnp.float32)


def _post_call(s, xin, root, bias, ic):
    return pl.pallas_call(
        functools.partial(_post_body, ic=ic),
        grid=(N // NB,),
        in_specs=[
            pl.BlockSpec((2, NB, 16), lambda i: (0, i, 0)),
            pl.BlockSpec((NB, xin.shape[1]), lambda i: (i, 0)),
            pl.BlockSpec(root.shape, lambda i: (0, 0)),
            pl.BlockSpec(bias.shape, lambda i: (0, 0)),
        ],
        out_specs=pl.BlockSpec((NB, 16), lambda i: (i, 0)),
        out_shape=jax.ShapeDtypeStruct((N, 16), jnp.float32),
    )(s, xin, root, bias)


def _post3_body(s3_ref, s1_ref, h2_ref, root_ref, bias_ref, idx_ref, out_ref,
                acc_ref, cnt_ref):
    i = pl.program_id(0)

    @pl.when(i == 0)
    def _():
        acc_ref[...] = jnp.zeros((B, 128), jnp.float32)
        cnt_ref[...] = jnp.zeros((B, 128), jnp.float32)

    s3 = s3_ref[0] + s3_ref[1]
    s1 = s1_ref[0] + s1_ref[1]
    cnt = jnp.maximum(s1[:, 10:11], 1.0)
    h3 = jnp.maximum(
        s3 / cnt + jnp.dot(h2_ref[...][:, 0:10], root_ref[...],
                           preferred_element_type=jnp.float32) + bias_ref[...],
        0.0)
    idx = idx_ref[0]
    oh = (lax.broadcasted_iota(jnp.int32, (B, NB), 0) == idx).astype(
        jnp.float32)
    acc_ref[...] += jnp.dot(oh, h3, preferred_element_type=jnp.float32)
    cnt_ref[...] += jnp.sum(oh, axis=1, keepdims=True)
    out_ref[...] = acc_ref[...] / jnp.maximum(cnt_ref[...], 1.0)


def _post3_call(s3, s1, h2, root, bias, idx3):
    return pl.pallas_call(
        _post3_body,
        grid=(N // NB,),
        in_specs=[
            pl.BlockSpec((2, NB, 128), lambda i: (0, i, 0)),
            pl.BlockSpec((2, NB, 16), lambda i: (0, i, 0)),
            pl.BlockSpec((NB, 16), lambda i: (i, 0)),
            pl.BlockSpec((10, 128), lambda i: (0, 0)),
            pl.BlockSpec((1, 128), lambda i: (0, 0)),
            pl.BlockSpec((1, 1, NB), lambda i: (i, 0, 0)),
        ],
        out_specs=pl.BlockSpec((B, 128), lambda i: (0, 0)),
        out_shape=jax.ShapeDtypeStruct((B, 128), jnp.float32),
        scratch_shapes=[
            pltpu.VMEM((B, 128), jnp.float32),
            pltpu.VMEM((B, 128), jnp.float32),
        ],
    )(s3, s1, h2, root, bias, idx3)


def _perm(a, i, o):
    lead = a.shape[:-1]
    return a.reshape(lead + (i, o)).swapaxes(-1, -2).reshape(lead + (i * o,))


def kernel(x, edge_index, edge_attr, cell_type, batch,
           emb1, Wh1, bh1, Wg1, bg1, root1, bias1,
           emb2, Wh2, bh2, Wg2, bg2, root2, bias2,
           emb3, Wh3, bh3, Wg3, bg3, root3, bias3):
    src2 = edge_index[0].reshape(E // CH, CH)
    dst2 = edge_index[1].reshape(E // CH, CH)
    ea = edge_attr

    def edge_t(emb_, wh_, bh_, wg_, bg_):
        return jnp.concatenate([
            emb_ * wh_[0:1], emb_ * wh_[1:2],
            emb_ * bh_[None, :] + bg_[None, :],
            wg_[0:1], wg_[1:2]], axis=0)

    t1 = edge_t(_perm(emb1, IN, 10), _perm(Wh1, IN, 10), _perm(bh1, IN, 10),
                _perm(Wg1, IN, 10), _perm(bg1, IN, 10))
    t2 = edge_t(_perm(emb2, 10, 10), _perm(Wh2, 10, 10), _perm(bh2, 10, 10),
                _perm(Wg2, 10, 10), _perm(bg2, 10, 10))
    t3 = edge_t(emb3, Wh3, bh3, Wg3, bg3)

    bf = jnp.bfloat16
    t1 = t1.astype(bf)
    t2 = t2.astype(bf)
    t3 = t3.astype(bf)
    seg1 = jnp.arange(1280) // 128
    s1m = (seg1[:, None] == jnp.arange(16)[None, :]).astype(bf)
    seg2 = jnp.arange(100) // 10
    s2m = (seg2[:, None] == jnp.arange(16)[None, :]).astype(bf)
    tb2 = (jnp.arange(10)[:, None] == (jnp.arange(100) % 10)[None, :]).astype(
        bf)
    tb3 = (jnp.arange(10)[:, None] == seg1[None, :]).astype(bf)

    bias1r = bias1[None, :]
    bias2r = bias2[None, :]
    bias3r = bias3[None, :]

    idx3 = ((cell_type == 1).astype(jnp.int32) * (batch + 1) - 1).reshape(
        N // NB, 1, NB)

    xs = _gather_rows(x, src2, 128)
    msg1 = _edge_call(_l1_body, EB1, 16, ea, xs, t1, s1m)
    s1 = _scatter_add(msg1, dst2, 16)
    h1 = _post_call(s1, x, root1, bias1r, ic=128)

    hs2 = _gather_rows(h1, src2, 16)
    msg2 = _edge_call(_l2_body, EB2, 16, ea, hs2, t2, tb2, s2m)
    s2 = _scatter_add(msg2, dst2, 16)
    h2 = _post_call(s2, h1, root2, bias2r, ic=10)

    hs3 = _gather_rows(h2, src2, 16)
    msg3 = _edge_call(_l3_body, EB3, 128, ea, hs3, t3, tb3)
    s3 = _scatter_add(msg3, dst2, 128)
    return _post3_call(s3, s1, h2, root3, bias3r, idx3)


# speedup vs baseline: 1.3906x; 1.0792x over previous
import functools

import jax
import jax.numpy as jnp
from jax import lax
from jax.experimental import pallas as pl
from jax.experimental.pallas import tpu as pltpu
from jax.experimental.pallas import tpu_sc as plsc

N = 10000
E = 320000
IN = 128
OUT = 128
NT = 25
B = 16

NC = 2
NS = 16
NW = NC * NS
EW = E // NW
CH = 80
NCH = EW // CH
RCH = 80
NRCH = N // RCH

EB1 = 4000
EB2 = 4000
EB3 = 4000
NB = 1000


def _sc_mesh():
    return plsc.VectorSubcoreMesh(core_axis_name="c", subcore_axis_name="s")


def _group(d):
    g = 5 if d > 16 else 25
    return g, g * CH, NCH // g


def _gather_rows(table, idx2, d):
    G, OCH, NG = _group(d)

    @functools.partial(
        pl.kernel,
        out_type=jax.ShapeDtypeStruct((E, d), jnp.float32),
        mesh=_sc_mesh(),
        compiler_params=pltpu.CompilerParams(use_tc_tiling_on_sc=False),
        scratch_types=[
            pltpu.VMEM((NCH, CH), jnp.int32),
            pltpu.VMEM((OCH, d), jnp.float32),
            pltpu.SemaphoreType.DMA,
        ],
    )
    def k(table_hbm, idx_hbm, out_hbm, idx_v, rows_v, sem):
        c = lax.axis_index("c")
        s = lax.axis_index("s")
        w = s * NC + c
        pltpu.sync_copy(idx_hbm.at[pl.ds(w * NCH, NCH)], idx_v)
        base = w * EW

        def body(g, carry):
            descs = []
            for j in range(G):
                descs.append(pltpu.async_copy(
                    table_hbm.at[idx_v.at[g * G + j]],
                    rows_v.at[pl.ds(j * CH, CH)], sem))
            for dsc in descs:
                dsc.wait()
            pltpu.sync_copy(rows_v, out_hbm.at[pl.ds(base + g * OCH, OCH)])
            return carry

        lax.fori_loop(0, NG, body, 0)

    return k(table, idx2)


def _scatter_add(msg, dst2, d):
    zeros = jnp.zeros((N, d), jnp.float32)
    G, OCH, NG = _group(d) if d <= 16 else (1, CH, NCH)

    @functools.partial(
        pl.kernel,
        out_type=jax.ShapeDtypeStruct((2, N, d), jnp.float32),
        mesh=_sc_mesh(),
        compiler_params=pltpu.CompilerParams(use_tc_tiling_on_sc=False),
        scratch_types=[
            pltpu.VMEM((NCH, CH), jnp.int32),
            pltpu.VMEM((OCH, d), jnp.float32),
            pltpu.VMEM_SHARED((N, d), jnp.float32),
            pltpu.SemaphoreType.DMA,
        ],
    )
    def k(msg_hbm, dst_hbm, zeros_hbm, out_hbm, idx_v, msg_v, accum, sem):
        c = lax.axis_index("c")
        s = lax.axis_index("s")
        w = s * NC + c
        pltpu.sync_copy(dst_hbm.at[pl.ds(w * NCH, NCH)], idx_v)

        def zbody(j, carry):
            r = (s + j * NS) * RCH
            pltpu.sync_copy(zeros_hbm.at[pl.ds(r, RCH)], accum.at[pl.ds(r, RCH)])
            return carry

        nj = (NRCH - s + NS - 1) // NS
        lax.fori_loop(0, nj, zbody, 0)
        plsc.subcore_barrier()

        base = w * EW

        def body(g, carry):
            pltpu.sync_copy(msg_hbm.at[pl.ds(base + g * OCH, OCH)], msg_v)
            descs = []
            for j in range(G):
                descs.append(pltpu.async_copy(
                    msg_v.at[pl.ds(j * CH, CH)],
                    accum.at[idx_v.at[g * G + j]], sem, add=True))
            for dsc in descs:
                dsc.wait()
            return carry

        lax.fori_loop(0, NG, body, 0)
        plsc.subcore_barrier()

        def obody(j, carry):
            r = (s + j * NS) * RCH
            pltpu.sync_copy(accum.at[pl.ds(r, RCH)], out_hbm.at[c, pl.ds(r, RCH)])
            return carry

        lax.fori_loop(0, nj, obody, 0)

    return k(msg, dst2, zeros)


BF = jnp.bfloat16


def _edge_w(ea, t_ref, eb, eo):
    et = ea[:, 0:1].astype(jnp.int32)
    oh = (et == lax.broadcasted_iota(jnp.int32, (eb, NT), 1)).astype(BF)
    f0 = ea[:, 1:2].astype(BF)
    f1 = ea[:, 2:3].astype(BF)
    m = jnp.concatenate([oh * f0, oh * f1, oh, f0, f1], axis=1)
    z = jnp.dot(m, t_ref[...], preferred_element_type=jnp.float32)
    return jnp.maximum(z, 0.0).astype(BF)


def _l1_body(ea_ref, xs_ref, t_ref, s_ref, out_ref):
    w = _edge_w(ea_ref[...], t_ref, EB1, 1280)
    xs = xs_ref[...].astype(BF)
    xs_bc = jnp.concatenate([xs] * 10, axis=1)
    out_ref[...] = jnp.dot(xs_bc * w, s_ref[...],
                           preferred_element_type=jnp.float32)
    out_ref[:, 10:11] = jnp.ones((EB1, 1), jnp.float32)


def _l2_body(ea_ref, hs_ref, t_ref, tb_ref, s_ref, out_ref):
    w = _edge_w(ea_ref[...], t_ref, EB2, 100)
    hs_bc = jnp.dot(hs_ref[...][:, 0:10].astype(BF), tb_ref[...],
                    preferred_element_type=jnp.float32).astype(BF)
    out_ref[...] = jnp.dot(hs_bc * w, s_ref[...],
                           preferred_element_type=jnp.float32)
    out_ref[:, 10:11] = jnp.ones((EB2, 1), jnp.float32)


def _l3_body(ea_ref, hs_ref, t_ref, tb_ref, out_ref):
    w = _edge_w(ea_ref[...], t_ref, EB3, 1280)
    hs_bc = jnp.dot(hs_ref[...][:, 0:10].astype(BF), tb_ref[...],
                    preferred_element_type=jnp.float32).astype(BF)
    prod = hs_bc * w
    acc = prod[:, 0:128]
    for i in range(1, 10):
        acc = acc + prod[:, i * 128:(i + 1) * 128]
    out_ref[...] = acc.astype(jnp.float32)


def _edge_call(body, eb, d_out, ea, xs, *mats):
    blk = lambda shape: pl.BlockSpec(shape, lambda i: (0,) * len(shape))
    return pl.pallas_call(
        body,
        grid=(E // eb,),
        in_specs=[
            pl.BlockSpec((eb, 3), lambda i: (i, 0)),
            pl.BlockSpec((eb, xs.shape[1]), lambda i: (i, 0)),
        ] + [blk(m.shape) for m in mats],
        out_specs=pl.BlockSpec((eb, d_out), lambda i: (i, 0)),
        out_shape=jax.ShapeDtypeStruct((E, d_out), jnp.float32),
    )(ea, xs, *mats)


def _post_body(s_ref, xin_ref, root_ref, bias_ref, out_ref, *, ic):
    s = s_ref[0] + s_ref[1]
    cnt = jnp.maximum(s[:, 10:11], 1.0)
    agg = s[:, 0:10] / cnt
    xin = xin_ref[...][:, 0:ic]
    z = agg + jnp.dot(xin, root_ref[...],
                      preferred_element_type=jnp.float32) + bias_ref[...]
    out_ref[:, 0:10] = jnp.maximum(z, 0.0)
    out_ref[:, 10:16] = jnp.zeros((NB, 6), jnp.float32)


def _post_call(s, xin, root, bias, ic):
    return pl.pallas_call(
        functools.partial(_post_body, ic=ic),
        grid=(N // NB,),
        in_specs=[
            pl.BlockSpec((2, NB, 16), lambda i: (0, i, 0)),
            pl.BlockSpec((NB, xin.shape[1]), lambda i: (i, 0)),
            pl.BlockSpec(root.shape, lambda i: (0, 0)),
            pl.BlockSpec(bias.shape, lambda i: (0, 0)),
        ],
        out_specs=pl.BlockSpec((NB, 16), lambda i: (i, 0)),
        out_shape=jax.ShapeDtypeStruct((N, 16), jnp.float32),
    )(s, xin, root, bias)


def _post3_body(s3_ref, s1_ref, h2_ref, root_ref, bias_ref, idx_ref, out_ref,
                acc_ref, cnt_ref):
    i = pl.program_id(0)

    @pl.when(i == 0)
    def _():
        acc_ref[...] = jnp.zeros((B, 128), jnp.float32)
        cnt_ref[...] = jnp.zeros((B, 128), jnp.float32)

    s3 = s3_ref[0] + s3_ref[1]
    s1 = s1_ref[0] + s1_ref[1]
    cnt = jnp.maximum(s1[:, 10:11], 1.0)
    h3 = jnp.maximum(
        s3 / cnt + jnp.dot(h2_ref[...][:, 0:10], root_ref[...],
                           preferred_element_type=jnp.float32) + bias_ref[...],
        0.0)
    idx = idx_ref[0]
    oh = (lax.broadcasted_iota(jnp.int32, (B, NB), 0) == idx).astype(
        jnp.float32)
    acc_ref[...] += jnp.dot(oh, h3, preferred_element_type=jnp.float32)
    cnt_ref[...] += jnp.sum(oh, axis=1, keepdims=True)
    out_ref[...] = acc_ref[...] / jnp.maximum(cnt_ref[...], 1.0)


def _post3_call(s3, s1, h2, root, bias, idx3):
    return pl.pallas_call(
        _post3_body,
        grid=(N // NB,),
        in_specs=[
            pl.BlockSpec((2, NB, 128), lambda i: (0, i, 0)),
            pl.BlockSpec((2, NB, 16), lambda i: (0, i, 0)),
            pl.BlockSpec((NB, 16), lambda i: (i, 0)),
            pl.BlockSpec((10, 128), lambda i: (0, 0)),
            pl.BlockSpec((1, 128), lambda i: (0, 0)),
            pl.BlockSpec((1, 1, NB), lambda i: (i, 0, 0)),
        ],
        out_specs=pl.BlockSpec((B, 128), lambda i: (0, 0)),
        out_shape=jax.ShapeDtypeStruct((B, 128), jnp.float32),
        scratch_shapes=[
            pltpu.VMEM((B, 128), jnp.float32),
            pltpu.VMEM((B, 128), jnp.float32),
        ],
    )(s3, s1, h2, root, bias, idx3)


def _perm(a, i, o):
    lead = a.shape[:-1]
    return a.reshape(lead + (i, o)).swapaxes(-1, -2).reshape(lead + (i * o,))


def kernel(x, edge_index, edge_attr, cell_type, batch,
           emb1, Wh1, bh1, Wg1, bg1, root1, bias1,
           emb2, Wh2, bh2, Wg2, bg2, root2, bias2,
           emb3, Wh3, bh3, Wg3, bg3, root3, bias3):
    src2 = edge_index[0].reshape(E // CH, CH)
    dst2 = edge_index[1].reshape(E // CH, CH)
    ea = edge_attr

    def edge_t(emb_, wh_, bh_, wg_, bg_):
        return jnp.concatenate([
            emb_ * wh_[0:1], emb_ * wh_[1:2],
            emb_ * bh_[None, :] + bg_[None, :],
            wg_[0:1], wg_[1:2]], axis=0)

    t1 = edge_t(_perm(emb1, IN, 10), _perm(Wh1, IN, 10), _perm(bh1, IN, 10),
                _perm(Wg1, IN, 10), _perm(bg1, IN, 10))
    t2 = edge_t(_perm(emb2, 10, 10), _perm(Wh2, 10, 10), _perm(bh2, 10, 10),
                _perm(Wg2, 10, 10), _perm(bg2, 10, 10))
    t3 = edge_t(emb3, Wh3, bh3, Wg3, bg3)

    bf = jnp.bfloat16
    t1 = t1.astype(bf)
    t2 = t2.astype(bf)
    t3 = t3.astype(bf)
    seg1 = jnp.arange(1280) // 128
    s1m = (seg1[:, None] == jnp.arange(16)[None, :]).astype(bf)
    seg2 = jnp.arange(100) // 10
    s2m = (seg2[:, None] == jnp.arange(16)[None, :]).astype(bf)
    tb2 = (jnp.arange(10)[:, None] == (jnp.arange(100) % 10)[None, :]).astype(
        bf)
    tb3 = (jnp.arange(10)[:, None] == seg1[None, :]).astype(bf)

    bias1r = bias1[None, :]
    bias2r = bias2[None, :]
    bias3r = bias3[None, :]

    idx3 = ((cell_type == 1).astype(jnp.int32) * (batch + 1) - 1).reshape(
        N // NB, 1, NB)

    xs = _gather_rows(x, src2, 128)
    msg1 = _edge_call(_l1_body, EB1, 16, ea, xs, t1, s1m)
    s1 = _scatter_add(msg1, dst2, 16)
    h1 = _post_call(s1, x, root1, bias1r, ic=128)

    hs2 = _gather_rows(h1, src2, 16)
    msg2 = _edge_call(_l2_body, EB2, 16, ea, hs2, t2, tb2, s2m)
    s2 = _scatter_add(msg2, dst2, 16)
    h2 = _post_call(s2, h1, root2, bias2r, ic=10)

    hs3 = _gather_rows(h2, src2, 16)
    msg3 = _edge_call(_l3_body, EB3, 128, ea, hs3, t3, tb3)
    s3 = _scatter_add(msg3, dst2, 128)
    return _post3_call(s3, s1, h2, root3, bias3r, idx3)


# speedup vs baseline: 1.4192x; 1.0206x over previous
import functools

import jax
import jax.numpy as jnp
from jax import lax
from jax.experimental import pallas as pl
from jax.experimental.pallas import tpu as pltpu
from jax.experimental.pallas import tpu_sc as plsc

N = 10000
E = 320000
IN = 128
OUT = 128
NT = 25
B = 16

NC = 2
NS = 16
NW = NC * NS
EW = E // NW
CH = 80
NCH = EW // CH
RCH = 80
NRCH = N // RCH

EB1 = 4000
EB2 = 4000
EB3 = 4000
NB = 1000


def _sc_mesh():
    return plsc.VectorSubcoreMesh(core_axis_name="c", subcore_axis_name="s")


def _group(d):
    g = 5 if d > 16 else 25
    return g, g * CH, NCH // g


def _gather_rows(table, idx2, d):
    G, OCH, NG = _group(d)

    @functools.partial(
        pl.kernel,
        out_type=jax.ShapeDtypeStruct((E, d), jnp.float32),
        mesh=_sc_mesh(),
        compiler_params=pltpu.CompilerParams(use_tc_tiling_on_sc=False),
        scratch_types=[
            pltpu.VMEM((NCH, CH), jnp.int32),
            pltpu.VMEM((OCH, d), jnp.float32),
            pltpu.SemaphoreType.DMA,
        ],
    )
    def k(table_hbm, idx_hbm, out_hbm, idx_v, rows_v, sem):
        c = lax.axis_index("c")
        s = lax.axis_index("s")
        w = s * NC + c
        pltpu.sync_copy(idx_hbm.at[pl.ds(w * NCH, NCH)], idx_v)
        base = w * EW

        def body(g, carry):
            descs = []
            for j in range(G):
                descs.append(pltpu.async_copy(
                    table_hbm.at[idx_v.at[g * G + j]],
                    rows_v.at[pl.ds(j * CH, CH)], sem))
            for dsc in descs:
                dsc.wait()
            pltpu.sync_copy(rows_v, out_hbm.at[pl.ds(base + g * OCH, OCH)])
            return carry

        lax.fori_loop(0, NG, body, 0)

    return k(table, idx2)


def _scatter_add(msg, dst2, d):
    zeros = jnp.zeros((N, d), jnp.float32)
    ring = d > 16
    G, OCH, NG = _group(d) if not ring else (1, CH, NCH)

    @functools.partial(
        pl.kernel,
        out_type=jax.ShapeDtypeStruct((2, N, d), jnp.float32),
        mesh=_sc_mesh(),
        compiler_params=pltpu.CompilerParams(use_tc_tiling_on_sc=False),
        scratch_types=[
            pltpu.VMEM((NCH, CH), jnp.int32),
            pltpu.VMEM((2, OCH, d), jnp.float32) if ring
            else pltpu.VMEM((OCH, d), jnp.float32),
            pltpu.VMEM_SHARED((N, d), jnp.float32),
            pltpu.SemaphoreType.DMA,
            pltpu.SemaphoreType.DMA,
        ],
    )
    def k(msg_hbm, dst_hbm, zeros_hbm, out_hbm, idx_v, msg_v, accum, sem,
          asem):
        c = lax.axis_index("c")
        s = lax.axis_index("s")
        w = s * NC + c
        pltpu.sync_copy(dst_hbm.at[pl.ds(w * NCH, NCH)], idx_v)

        def zbody(j, carry):
            r = (s + j * NS) * RCH
            pltpu.sync_copy(zeros_hbm.at[pl.ds(r, RCH)], accum.at[pl.ds(r, RCH)])
            return carry

        nj = (NRCH - s + NS - 1) // NS
        lax.fori_loop(0, nj, zbody, 0)
        plsc.subcore_barrier()

        base = w * EW

        if ring:
            pltpu.async_copy(msg_hbm.at[pl.ds(base, CH)], msg_v.at[0], sem)

            def body(g, carry):
                b = g % 2
                pltpu.make_async_copy(
                    msg_hbm.at[pl.ds(base + g * CH, CH)], msg_v.at[b],
                    sem).wait()

                @pl.when(g >= 1)
                def _():
                    pltpu.make_async_copy(
                        msg_v.at[1 - b], accum.at[idx_v.at[g - 1]],
                        asem).wait()

                @pl.when(g + 1 < NCH)
                def _():
                    pltpu.async_copy(
                        msg_hbm.at[pl.ds(base + (g + 1) * CH, CH)],
                        msg_v.at[1 - b], sem)

                pltpu.async_copy(msg_v.at[b], accum.at[idx_v.at[g]], asem,
                                 add=True)
                return carry

            lax.fori_loop(0, NCH, body, 0)
            pltpu.make_async_copy(
                msg_v.at[(NCH - 1) % 2], accum.at[idx_v.at[NCH - 1]],
                asem).wait()
        else:
            def body(g, carry):
                pltpu.sync_copy(msg_hbm.at[pl.ds(base + g * OCH, OCH)], msg_v)
                descs = []
                for j in range(G):
                    descs.append(pltpu.async_copy(
                        msg_v.at[pl.ds(j * CH, CH)],
                        accum.at[idx_v.at[g * G + j]], sem, add=True))
                for dsc in descs:
                    dsc.wait()
                return carry

            lax.fori_loop(0, NG, body, 0)
        plsc.subcore_barrier()

        def obody(j, carry):
            r = (s + j * NS) * RCH
            pltpu.sync_copy(accum.at[pl.ds(r, RCH)], out_hbm.at[c, pl.ds(r, RCH)])
            return carry

        lax.fori_loop(0, nj, obody, 0)

    return k(msg, dst2, zeros)


BF = jnp.bfloat16


def _edge_w(ea, t_ref, eb, eo):
    et = ea[:, 0:1].astype(jnp.int32)
    oh = (et == lax.broadcasted_iota(jnp.int32, (eb, NT), 1)).astype(BF)
    f0 = ea[:, 1:2].astype(BF)
    f1 = ea[:, 2:3].astype(BF)
    m = jnp.concatenate([oh * f0, oh * f1, oh, f0, f1], axis=1)
    z = jnp.dot(m, t_ref[...], preferred_element_type=jnp.float32)
    return jnp.maximum(z, 0.0).astype(BF)


def _l1_body(ea_ref, xs_ref, t_ref, s_ref, out_ref):
    w = _edge_w(ea_ref[...], t_ref, EB1, 1280)
    xs = xs_ref[...].astype(BF)
    xs_bc = jnp.concatenate([xs] * 10, axis=1)
    out_ref[...] = jnp.dot(xs_bc * w, s_ref[...],
                           preferred_element_type=jnp.float32)
    out_ref[:, 10:11] = jnp.ones((EB1, 1), jnp.float32)


def _l2_body(ea_ref, hs_ref, t_ref, tb_ref, s_ref, out_ref):
    w = _edge_w(ea_ref[...], t_ref, EB2, 100)
    hs_bc = jnp.dot(hs_ref[...][:, 0:10].astype(BF), tb_ref[...],
                    preferred_element_type=jnp.float32).astype(BF)
    out_ref[...] = jnp.dot(hs_bc * w, s_ref[...],
                           preferred_element_type=jnp.float32)
    out_ref[:, 10:11] = jnp.ones((EB2, 1), jnp.float32)


def _l3_body(ea_ref, hs_ref, t_ref, tb_ref, out_ref):
    w = _edge_w(ea_ref[...], t_ref, EB3, 1280)
    hs_bc = jnp.dot(hs_ref[...][:, 0:10].astype(BF), tb_ref[...],
                    preferred_element_type=jnp.float32).astype(BF)
    prod = hs_bc * w
    acc = prod[:, 0:128]
    for i in range(1, 10):
        acc = acc + prod[:, i * 128:(i + 1) * 128]
    out_ref[...] = acc.astype(jnp.float32)


def _edge_call(body, eb, d_out, ea, xs, *mats):
    blk = lambda shape: pl.BlockSpec(shape, lambda i: (0,) * len(shape))
    return pl.pallas_call(
        body,
        grid=(E // eb,),
        in_specs=[
            pl.BlockSpec((eb, 3), lambda i: (i, 0)),
            pl.BlockSpec((eb, xs.shape[1]), lambda i: (i, 0)),
        ] + [blk(m.shape) for m in mats],
        out_specs=pl.BlockSpec((eb, d_out), lambda i: (i, 0)),
        out_shape=jax.ShapeDtypeStruct((E, d_out), jnp.float32),
    )(ea, xs, *mats)


def _post_body(s_ref, xin_ref, root_ref, bias_ref, out_ref, *, ic):
    s = s_ref[0] + s_ref[1]
    cnt = jnp.maximum(s[:, 10:11], 1.0)
    agg = s[:, 0:10] / cnt
    xin = xin_ref[...][:, 0:ic]
    z = agg + jnp.dot(xin, root_ref[...],
                      preferred_element_type=jnp.float32) + bias_ref[...]
    out_ref[:, 0:10] = jnp.maximum(z, 0.0)
    out_ref[:, 10:16] = jnp.zeros((NB, 6), jnp.float32)


def _post_call(s, xin, root, bias, ic):
    return pl.pallas_call(
        functools.partial(_post_body, ic=ic),
        grid=(N // NB,),
        in_specs=[
            pl.BlockSpec((2, NB, 16), lambda i: (0, i, 0)),
            pl.BlockSpec((NB, xin.shape[1]), lambda i: (i, 0)),
            pl.BlockSpec(root.shape, lambda i: (0, 0)),
            pl.BlockSpec(bias.shape, lambda i: (0, 0)),
        ],
        out_specs=pl.BlockSpec((NB, 16), lambda i: (i, 0)),
        out_shape=jax.ShapeDtypeStruct((N, 16), jnp.float32),
    )(s, xin, root, bias)


def _post3_body(s3_ref, s1_ref, h2_ref, root_ref, bias_ref, idx_ref, out_ref,
                acc_ref, cnt_ref):
    i = pl.program_id(0)

    @pl.when(i == 0)
    def _():
        acc_ref[...] = jnp.zeros((B, 128), jnp.float32)
        cnt_ref[...] = jnp.zeros((B, 128), jnp.float32)

    s3 = s3_ref[0] + s3_ref[1]
    s1 = s1_ref[0] + s1_ref[1]
    cnt = jnp.maximum(s1[:, 10:11], 1.0)
    h3 = jnp.maximum(
        s3 / cnt + jnp.dot(h2_ref[...][:, 0:10], root_ref[...],
                           preferred_element_type=jnp.float32) + bias_ref[...],
        0.0)
    idx = idx_ref[0]
    oh = (lax.broadcasted_iota(jnp.int32, (B, NB), 0) == idx).astype(
        jnp.float32)
    acc_ref[...] += jnp.dot(oh, h3, preferred_element_type=jnp.float32)
    cnt_ref[...] += jnp.sum(oh, axis=1, keepdims=True)
    out_ref[...] = acc_ref[...] / jnp.maximum(cnt_ref[...], 1.0)


def _post3_call(s3, s1, h2, root, bias, idx3):
    return pl.pallas_call(
        _post3_body,
        grid=(N // NB,),
        in_specs=[
            pl.BlockSpec((2, NB, 128), lambda i: (0, i, 0)),
            pl.BlockSpec((2, NB, 16), lambda i: (0, i, 0)),
            pl.BlockSpec((NB, 16), lambda i: (i, 0)),
            pl.BlockSpec((10, 128), lambda i: (0, 0)),
            pl.BlockSpec((1, 128), lambda i: (0, 0)),
            pl.BlockSpec((1, 1, NB), lambda i: (i, 0, 0)),
        ],
        out_specs=pl.BlockSpec((B, 128), lambda i: (0, 0)),
        out_shape=jax.ShapeDtypeStruct((B, 128), jnp.float32),
        scratch_shapes=[
            pltpu.VMEM((B, 128), jnp.float32),
            pltpu.VMEM((B, 128), jnp.float32),
        ],
    )(s3, s1, h2, root, bias, idx3)


def _perm(a, i, o):
    lead = a.shape[:-1]
    return a.reshape(lead + (i, o)).swapaxes(-1, -2).reshape(lead + (i * o,))


def kernel(x, edge_index, edge_attr, cell_type, batch,
           emb1, Wh1, bh1, Wg1, bg1, root1, bias1,
           emb2, Wh2, bh2, Wg2, bg2, root2, bias2,
           emb3, Wh3, bh3, Wg3, bg3, root3, bias3):
    src2 = edge_index[0].reshape(E // CH, CH)
    dst2 = edge_index[1].reshape(E // CH, CH)
    ea = edge_attr

    def edge_t(emb_, wh_, bh_, wg_, bg_):
        return jnp.concatenate([
            emb_ * wh_[0:1], emb_ * wh_[1:2],
            emb_ * bh_[None, :] + bg_[None, :],
            wg_[0:1], wg_[1:2]], axis=0)

    t1 = edge_t(_perm(emb1, IN, 10), _perm(Wh1, IN, 10), _perm(bh1, IN, 10),
                _perm(Wg1, IN, 10), _perm(bg1, IN, 10))
    t2 = edge_t(_perm(emb2, 10, 10), _perm(Wh2, 10, 10), _perm(bh2, 10, 10),
                _perm(Wg2, 10, 10), _perm(bg2, 10, 10))
    t3 = edge_t(emb3, Wh3, bh3, Wg3, bg3)

    bf = jnp.bfloat16
    t1 = t1.astype(bf)
    t2 = t2.astype(bf)
    t3 = t3.astype(bf)
    seg1 = jnp.arange(1280) // 128
    s1m = (seg1[:, None] == jnp.arange(16)[None, :]).astype(bf)
    seg2 = jnp.arange(100) // 10
    s2m = (seg2[:, None] == jnp.arange(16)[None, :]).astype(bf)
    tb2 = (jnp.arange(10)[:, None] == (jnp.arange(100) % 10)[None, :]).astype(
        bf)
    tb3 = (jnp.arange(10)[:, None] == seg1[None, :]).astype(bf)

    bias1r = bias1[None, :]
    bias2r = bias2[None, :]
    bias3r = bias3[None, :]

    idx3 = ((cell_type == 1).astype(jnp.int32) * (batch + 1) - 1).reshape(
        N // NB, 1, NB)

    xs = _gather_rows(x, src2, 128)
    msg1 = _edge_call(_l1_body, EB1, 16, ea, xs, t1, s1m)
    s1 = _scatter_add(msg1, dst2, 16)
    h1 = _post_call(s1, x, root1, bias1r, ic=128)

    hs2 = _gather_rows(h1, src2, 16)
    msg2 = _edge_call(_l2_body, EB2, 16, ea, hs2, t2, tb2, s2m)
    s2 = _scatter_add(msg2, dst2, 16)
    h2 = _post_call(s2, h1, root2, bias2r, ic=10)

    hs3 = _gather_rows(h2, src2, 16)
    msg3 = _edge_call(_l3_body, EB3, 128, ea, hs3, t3, tb3)
    s3 = _scatter_add(msg3, dst2, 128)
    return _post3_call(s3, s1, h2, root3, bias3r, idx3)
